# baseline (speedup 1.0000x reference)
import functools

import jax
import jax.numpy as jnp
from jax import lax
from jax.experimental import pallas as pl
from jax.experimental.pallas import tpu as pltpu
from jax.experimental.pallas import tpu_sc as plsc

NE = 16
D = 1024
NT = 8192
NA = 2 * NT
TBLK = 512
BLK = 128
NBLK = NA // BLK + NE
PAD = NBLK * BLK

NC, NS = 2, 16
NW = NC * NS
TPW = NT // NW
CH = 32
NCH = TPW // CH
CHE = 16
NCHE = TPW // CHE
GW = 128


def _router_body(x_ref, rw_ref, rb_ref, i0_ref, i1_ref, g0_ref, g1_ref):
    xb = x_ref[...]
    logits = lax.dot_general(xb, rw_ref[...], (((1,), (1,)), ((), ())),
                             preferred_element_type=jnp.float32) + rb_ref[0][None, :]
    probs = jax.nn.softmax(logits, axis=-1)
    cols = lax.broadcasted_iota(jnp.int32, probs.shape, 1)
    m0 = jnp.max(probs, axis=-1, keepdims=True)
    i0 = jnp.argmax(probs, axis=-1)
    masked = jnp.where(cols == i0[:, None], -jnp.inf, probs)
    m1 = jnp.max(masked, axis=-1, keepdims=True)
    i1 = jnp.argmax(masked, axis=-1)
    i0_ref[...] = i0.reshape(1, 1, TBLK)
    i1_ref[...] = i1.reshape(1, 1, TBLK)
    g0_ref[...] = jnp.broadcast_to(m0, (TBLK, GW))
    g1_ref[...] = jnp.broadcast_to(m1, (TBLK, GW))


def _router(x, router_W, router_b):
    nblk = NT // TBLK
    return pl.pallas_call(
        _router_body,
        grid=(nblk,),
        in_specs=[
            pl.BlockSpec((TBLK, D), lambda t: (t, 0)),
            pl.BlockSpec((NE, D), lambda t: (0, 0)),
            pl.BlockSpec((1, NE), lambda t: (0, 0)),
        ],
        out_specs=[
            pl.BlockSpec((1, 1, TBLK), lambda t: (t, 0, 0)),
            pl.BlockSpec((1, 1, TBLK), lambda t: (t, 0, 0)),
            pl.BlockSpec((TBLK, GW), lambda t: (t, 0)),
            pl.BlockSpec((TBLK, GW), lambda t: (t, 0)),
        ],
        out_shape=[
            jax.ShapeDtypeStruct((nblk, 1, TBLK), jnp.int32),
            jax.ShapeDtypeStruct((nblk, 1, TBLK), jnp.int32),
            jax.ShapeDtypeStruct((NT, GW), jnp.float32),
            jax.ShapeDtypeStruct((NT, GW), jnp.float32),
        ],
    )(x, router_W, router_b.reshape(1, NE))


_G = 128


def _book_body(e0_ref, e1_ref, p0_ref, p1_ref, be_ref):
    ea = jnp.concatenate([e0_ref[...], e1_ref[...]], axis=0)
    rows = lax.broadcasted_iota(jnp.int32, (_G, _G), 0)
    colsq = lax.broadcasted_iota(jnp.int32, (_G, _G), 1)
    U = (rows <= colsq).astype(jnp.float32)
    Lx = (colsq < rows).astype(jnp.float32)

    onehots, totals = [], []
    for e in range(NE):
        ohf = (ea == e).astype(jnp.float32)
        C = lax.dot_general(ohf, U, (((1,), (0,)), ((), ())),
                            preferred_element_type=jnp.float32)
        S = C[:, _G - 1:_G]
        P = lax.dot_general(Lx, S, (((1,), (0,)), ((), ())),
                            preferred_element_type=jnp.float32)
        rank = (P + C).astype(jnp.int32) - 1
        onehots.append((ohf.astype(jnp.int32), rank))
        totals.append(jnp.sum(S).astype(jnp.int32))

    pos = jnp.zeros((_G, _G), jnp.int32)
    start_blocks = []
    sbk = jnp.int32(0)
    for e in range(NE):
        start_blocks.append(sbk)
        oh, rank = onehots[e]
        pos = pos + oh * (sbk * BLK + rank)
        sbk = sbk + (totals[e] + (BLK - 1)) // BLK

    p0_ref[...] = pos[:_G // 2]
    p1_ref[...] = pos[_G // 2:]

    biota = (lax.broadcasted_iota(jnp.int32, (8, 128), 1)
             + 128 * lax.broadcasted_iota(jnp.int32, (8, 128), 0))
    be = jnp.zeros((8, 128), jnp.int32)
    for e in range(NE):
        be = be + (biota >= start_blocks[e]).astype(jnp.int32)
    be_ref[...] = be - 1


def _bookkeeping(e0_2d, e1_2d):
    return pl.pallas_call(
        _book_body,
        out_shape=[
            jax.ShapeDtypeStruct((_G // 2, _G), jnp.int32),
            jax.ShapeDtypeStruct((_G // 2, _G), jnp.int32),
            jax.ShapeDtypeStruct((8, 128), jnp.int32),
        ],
    )(e0_2d, e1_2d)


def _dispatch(x, p0r, p1r, g0b, g1b):
    mesh = plsc.VectorSubcoreMesh(core_axis_name="c", subcore_axis_name="s")

    @functools.partial(
        pl.kernel, mesh=mesh,
        out_type=[
            jax.ShapeDtypeStruct((PAD, D), jnp.float32),
            jax.ShapeDtypeStruct((PAD, GW), jnp.float32),
        ],
        scratch_types=[
            pltpu.VMEM((CH, D), jnp.float32),
            pltpu.VMEM((CH, D), jnp.float32),
            pltpu.VMEM((CH, GW), jnp.float32),
            pltpu.VMEM((CH, GW), jnp.float32),
            pltpu.VMEM((CH, GW), jnp.float32),
            pltpu.VMEM((CH, GW), jnp.float32),
            pltpu.VMEM((NCH, CH), jnp.int32),
            pltpu.VMEM((NCH, CH), jnp.int32),
            pltpu.SemaphoreType.DMA,
            pltpu.SemaphoreType.DMA,
            pltpu.SemaphoreType.DMA,
            pltpu.SemaphoreType.DMA,
        ],
    )
    def body(x_hbm, p0_hbm, p1_hbm, g0_hbm, g1_hbm, xs_hbm, gs_hbm,
             xb0, xb1, ga0, ga1, gb0, gb1, i0all, i1all,
             ldsem0, ldsem1, scsem, gssem):
        wid = lax.axis_index("s") * NC + lax.axis_index("c")
        t0 = wid * TPW
        r0 = wid * NCH
        pltpu.sync_copy(p0_hbm.at[pl.ds(r0, NCH)], i0all)
        pltpu.sync_copy(p1_hbm.at[pl.ds(r0, NCH)], i1all)

        xbufs = (xb0, xb1)
        g0bufs = (ga0, ga1)
        g1bufs = (gb0, gb1)
        ldsems = (ldsem0, ldsem1)
        loads = [None, None]
        loads[0] = [
            pltpu.async_copy(x_hbm.at[pl.ds(t0, CH)], xb0, ldsem0),
            pltpu.async_copy(g0_hbm.at[pl.ds(t0, CH)], ga0, ldsem0),
            pltpu.async_copy(g1_hbm.at[pl.ds(t0, CH)], gb0, ldsem0),
        ]
        pending = []
        for j in range(NCH):
            b = j % 2
            for ld in loads[b]:
                ld.wait()
            if pending:
                for c in pending.pop(0):
                    c.wait()
            if j + 1 < NCH:
                o = t0 + (j + 1) * CH
                loads[1 - b] = [
                    pltpu.async_copy(x_hbm.at[pl.ds(o, CH)],
                                     xbufs[1 - b], ldsems[1 - b]),
                    pltpu.async_copy(g0_hbm.at[pl.ds(o, CH)],
                                     g0bufs[1 - b], ldsems[1 - b]),
                    pltpu.async_copy(g1_hbm.at[pl.ds(o, CH)],
                                     g1bufs[1 - b], ldsems[1 - b]),
                ]
            pending.append([
                pltpu.async_copy(xbufs[b], xs_hbm.at[i0all.at[j]], scsem),
                pltpu.async_copy(xbufs[b], xs_hbm.at[i1all.at[j]], scsem),
                pltpu.async_copy(g0bufs[b], gs_hbm.at[i0all.at[j]], gssem),
                pltpu.async_copy(g1bufs[b], gs_hbm.at[i1all.at[j]], gssem),
            ])
        for grp in pending:
            for c in grp:
                c.wait()

    return body(x, p0r, p1r, g0b, g1b)


def _gmm_body(be_ref, xs_ref, w_ref, b_ref, gs_ref, y_ref):
    del be_ref
    acc = lax.dot_general(
        xs_ref[...], w_ref[0], (((1,), (1,)), ((), ())),
        preferred_element_type=jnp.float32) + b_ref[0]
    y_ref[...] = acc * gs_ref[:, 0:1]


def _grouped_matmul(be, xs, expert_W, expert_b, gs):
    grid_spec = pltpu.PrefetchScalarGridSpec(
        num_scalar_prefetch=1,
        grid=(NBLK,),
        in_specs=[
            pl.BlockSpec((BLK, D), lambda i, be_s: (i, 0)),
            pl.BlockSpec((1, D, D), lambda i, be_s: (be_s[i], 0, 0)),
            pl.BlockSpec((1, 1, D), lambda i, be_s: (be_s[i], 0, 0)),
            pl.BlockSpec((BLK, GW), lambda i, be_s: (i, 0)),
        ],
        out_specs=pl.BlockSpec((BLK, D), lambda i, be_s: (i, 0)),
    )
    return pl.pallas_call(
        _gmm_body,
        grid_spec=grid_spec,
        out_shape=jax.ShapeDtypeStruct((PAD, D), jnp.float32),
    )(be, xs, expert_W, expert_b.reshape(NE, 1, D), gs)


def _combine(ys, pcat):
    mesh = plsc.VectorSubcoreMesh(core_axis_name="c", subcore_axis_name="s")

    @functools.partial(
        pl.kernel, mesh=mesh,
        out_type=jax.ShapeDtypeStruct((NT, D), jnp.float32),
        scratch_types=[
            pltpu.VMEM((2 * CHE, D), jnp.float32),
            pltpu.VMEM((2 * CHE, D), jnp.float32),
            pltpu.VMEM((CHE, D), jnp.float32),
            pltpu.VMEM((CHE, D), jnp.float32),
            pltpu.VMEM((NCHE, 2 * CHE), jnp.int32),
            pltpu.SemaphoreType.DMA,
            pltpu.SemaphoreType.DMA,
            pltpu.SemaphoreType.DMA,
            pltpu.SemaphoreType.DMA,
        ],
    )
    def body(y_hbm, pc_hbm, out_hbm, rb0, rb1, ob0, ob1,
             pcall, gsem0, gsem1, osem0, osem1):
        wid = lax.axis_index("s") * NC + lax.axis_index("c")
        t0 = wid * TPW
        r0 = wid * NCHE
        pltpu.sync_copy(pc_hbm.at[pl.ds(r0, NCHE)], pcall)

        rbufs = (rb0, rb1)
        obufs = (ob0, ob1)
        gsems = (gsem0, gsem1)
        osems = (osem0, osem1)
        gathers = [None, None]
        owrite = [None, None]
        gathers[0] = pltpu.async_copy(y_hbm.at[pcall.at[0]], rb0, gsem0)
        for j in range(NCHE):
            b = j % 2
            gathers[b].wait()
            if j + 1 < NCHE:
                gathers[1 - b] = pltpu.async_copy(
                    y_hbm.at[pcall.at[j + 1]], rbufs[1 - b], gsems[1 - b])
            rb = rbufs[b]
            ob = obufs[b]
            if owrite[b] is not None:
                owrite[b].wait()
            def row(r, carry):
                def col(c, carry2):
                    sl = pl.ds(c * 16, 16)
                    ob[r, sl] = rb[r, sl] + rb[r + CHE, sl]
                    return carry2

                lax.fori_loop(0, D // 16, col, 0)
                return carry

            lax.fori_loop(0, CHE, row, 0)
            owrite[b] = pltpu.async_copy(
                ob, out_hbm.at[pl.ds(t0 + j * CHE, CHE)], osems[b])
        for ow in owrite:
            if ow is not None:
                ow.wait()

    return body(ys, pcat)


def kernel(x, expert_W, expert_b, router_W, router_b):
    i0, i1, g0b, g1b = _router(x, router_W, router_b)
    e0_2d = i0.reshape(NT).reshape(_G // 2, _G)
    e1_2d = i1.reshape(NT).reshape(_G // 2, _G)
    p0_2d, p1_2d, be2d = _bookkeeping(e0_2d, e1_2d)
    p0 = p0_2d.reshape(NT)
    p1 = p1_2d.reshape(NT)
    be = be2d.reshape(1024)[:NBLK]
    p0r = p0.reshape(NT // CH, CH)
    p1r = p1.reshape(NT // CH, CH)
    xs, gs = _dispatch(x, p0r, p1r, g0b, g1b)
    ys = _grouped_matmul(be, xs, expert_W, expert_b, gs)
    pcat = jnp.concatenate(
        [p0.reshape(NT // CHE, CHE), p1.reshape(NT // CHE, CHE)], axis=1)
    return _combine(ys, pcat)


# speedup vs baseline: 1.1988x; 1.1988x over previous
import functools

import jax
import jax.numpy as jnp
from jax import lax
from jax.experimental import pallas as pl
from jax.experimental.pallas import tpu as pltpu
from jax.experimental.pallas import tpu_sc as plsc

NE = 16
D = 1024
NT = 8192
NA = 2 * NT
TBLK = 512
BLK = 256
NBLK = NA // BLK + NE
PAD = NBLK * BLK

NC, NS = 2, 16
NW = NC * NS
TPW = NT // NW
CH = 32
NCH = TPW // CH
CHE = 16
NCHE = TPW // CHE
GW = 128


def _router_body(x_ref, rw_ref, rb_ref, i0_ref, i1_ref, g0_ref, g1_ref):
    xb = x_ref[...]
    logits = lax.dot_general(xb, rw_ref[...], (((1,), (1,)), ((), ())),
                             preferred_element_type=jnp.float32) + rb_ref[0][None, :]
    probs = jax.nn.softmax(logits, axis=-1)
    cols = lax.broadcasted_iota(jnp.int32, probs.shape, 1)
    m0 = jnp.max(probs, axis=-1, keepdims=True)
    i0 = jnp.argmax(probs, axis=-1)
    masked = jnp.where(cols == i0[:, None], -jnp.inf, probs)
    m1 = jnp.max(masked, axis=-1, keepdims=True)
    i1 = jnp.argmax(masked, axis=-1)
    i0_ref[...] = i0.reshape(1, 1, TBLK)
    i1_ref[...] = i1.reshape(1, 1, TBLK)
    g0_ref[...] = jnp.broadcast_to(m0, (TBLK, GW))
    g1_ref[...] = jnp.broadcast_to(m1, (TBLK, GW))


def _router(x, router_W, router_b):
    nblk = NT // TBLK
    return pl.pallas_call(
        _router_body,
        grid=(nblk,),
        in_specs=[
            pl.BlockSpec((TBLK, D), lambda t: (t, 0)),
            pl.BlockSpec((NE, D), lambda t: (0, 0)),
            pl.BlockSpec((1, NE), lambda t: (0, 0)),
        ],
        out_specs=[
            pl.BlockSpec((1, 1, TBLK), lambda t: (t, 0, 0)),
            pl.BlockSpec((1, 1, TBLK), lambda t: (t, 0, 0)),
            pl.BlockSpec((TBLK, GW), lambda t: (t, 0)),
            pl.BlockSpec((TBLK, GW), lambda t: (t, 0)),
        ],
        out_shape=[
            jax.ShapeDtypeStruct((nblk, 1, TBLK), jnp.int32),
            jax.ShapeDtypeStruct((nblk, 1, TBLK), jnp.int32),
            jax.ShapeDtypeStruct((NT, GW), jnp.float32),
            jax.ShapeDtypeStruct((NT, GW), jnp.float32),
        ],
    )(x, router_W, router_b.reshape(1, NE))


_G = 128


def _book_body(e0_ref, e1_ref, p0_ref, p1_ref, be_ref):
    ea = jnp.concatenate([e0_ref[...], e1_ref[...]], axis=0)
    rows = lax.broadcasted_iota(jnp.int32, (_G, _G), 0)
    colsq = lax.broadcasted_iota(jnp.int32, (_G, _G), 1)
    U = (rows <= colsq).astype(jnp.float32)
    Lx = (colsq < rows).astype(jnp.float32)

    onehots, totals = [], []
    for e in range(NE):
        ohf = (ea == e).astype(jnp.float32)
        C = lax.dot_general(ohf, U, (((1,), (0,)), ((), ())),
                            preferred_element_type=jnp.float32)
        S = C[:, _G - 1:_G]
        P = lax.dot_general(Lx, S, (((1,), (0,)), ((), ())),
                            preferred_element_type=jnp.float32)
        rank = (P + C).astype(jnp.int32) - 1
        onehots.append((ohf.astype(jnp.int32), rank))
        totals.append(jnp.sum(S).astype(jnp.int32))

    pos = jnp.zeros((_G, _G), jnp.int32)
    start_blocks = []
    sbk = jnp.int32(0)
    for e in range(NE):
        start_blocks.append(sbk)
        oh, rank = onehots[e]
        pos = pos + oh * (sbk * BLK + rank)
        sbk = sbk + (totals[e] + (BLK - 1)) // BLK

    p0_ref[...] = pos[:_G // 2]
    p1_ref[...] = pos[_G // 2:]

    biota = (lax.broadcasted_iota(jnp.int32, (8, 128), 1)
             + 128 * lax.broadcasted_iota(jnp.int32, (8, 128), 0))
    be = jnp.zeros((8, 128), jnp.int32)
    for e in range(NE):
        be = be + (biota >= start_blocks[e]).astype(jnp.int32)
    be_ref[...] = be - 1


def _bookkeeping(e0_2d, e1_2d):
    return pl.pallas_call(
        _book_body,
        out_shape=[
            jax.ShapeDtypeStruct((_G // 2, _G), jnp.int32),
            jax.ShapeDtypeStruct((_G // 2, _G), jnp.int32),
            jax.ShapeDtypeStruct((8, 128), jnp.int32),
        ],
    )(e0_2d, e1_2d)


def _dispatch(x, p0r, p1r, g0b, g1b):
    mesh = plsc.VectorSubcoreMesh(core_axis_name="c", subcore_axis_name="s")

    @functools.partial(
        pl.kernel, mesh=mesh,
        out_type=[
            jax.ShapeDtypeStruct((PAD, D), jnp.float32),
            jax.ShapeDtypeStruct((PAD, GW), jnp.float32),
        ],
        scratch_types=[
            pltpu.VMEM((CH, D), jnp.float32),
            pltpu.VMEM((CH, D), jnp.float32),
            pltpu.VMEM((CH, GW), jnp.float32),
            pltpu.VMEM((CH, GW), jnp.float32),
            pltpu.VMEM((CH, GW), jnp.float32),
            pltpu.VMEM((CH, GW), jnp.float32),
            pltpu.VMEM((NCH, CH), jnp.int32),
            pltpu.VMEM((NCH, CH), jnp.int32),
            pltpu.SemaphoreType.DMA,
            pltpu.SemaphoreType.DMA,
            pltpu.SemaphoreType.DMA,
            pltpu.SemaphoreType.DMA,
        ],
    )
    def body(x_hbm, p0_hbm, p1_hbm, g0_hbm, g1_hbm, xs_hbm, gs_hbm,
             xb0, xb1, ga0, ga1, gb0, gb1, i0all, i1all,
             ldsem0, ldsem1, scsem, gssem):
        wid = lax.axis_index("s") * NC + lax.axis_index("c")
        t0 = wid * TPW
        r0 = wid * NCH
        pltpu.sync_copy(p0_hbm.at[pl.ds(r0, NCH)], i0all)
        pltpu.sync_copy(p1_hbm.at[pl.ds(r0, NCH)], i1all)

        xbufs = (xb0, xb1)
        g0bufs = (ga0, ga1)
        g1bufs = (gb0, gb1)
        ldsems = (ldsem0, ldsem1)
        loads = [None, None]
        loads[0] = [
            pltpu.async_copy(x_hbm.at[pl.ds(t0, CH)], xb0, ldsem0),
            pltpu.async_copy(g0_hbm.at[pl.ds(t0, CH)], ga0, ldsem0),
            pltpu.async_copy(g1_hbm.at[pl.ds(t0, CH)], gb0, ldsem0),
        ]
        pending = []
        for j in range(NCH):
            b = j % 2
            for ld in loads[b]:
                ld.wait()
            if pending:
                for c in pending.pop(0):
                    c.wait()
            if j + 1 < NCH:
                o = t0 + (j + 1) * CH
                loads[1 - b] = [
                    pltpu.async_copy(x_hbm.at[pl.ds(o, CH)],
                                     xbufs[1 - b], ldsems[1 - b]),
                    pltpu.async_copy(g0_hbm.at[pl.ds(o, CH)],
                                     g0bufs[1 - b], ldsems[1 - b]),
                    pltpu.async_copy(g1_hbm.at[pl.ds(o, CH)],
                                     g1bufs[1 - b], ldsems[1 - b]),
                ]
            pending.append([
                pltpu.async_copy(xbufs[b], xs_hbm.at[i0all.at[j]], scsem),
                pltpu.async_copy(xbufs[b], xs_hbm.at[i1all.at[j]], scsem),
                pltpu.async_copy(g0bufs[b], gs_hbm.at[i0all.at[j]], gssem),
                pltpu.async_copy(g1bufs[b], gs_hbm.at[i1all.at[j]], gssem),
            ])
        for grp in pending:
            for c in grp:
                c.wait()

    return body(x, p0r, p1r, g0b, g1b)


def _gmm_body(be_ref, xs_ref, w_ref, b_ref, gs_ref, y_ref):
    del be_ref
    acc = lax.dot_general(
        xs_ref[...], w_ref[0], (((1,), (1,)), ((), ())),
        preferred_element_type=jnp.float32) + b_ref[0]
    y_ref[...] = acc * gs_ref[:, 0:1]


def _grouped_matmul(be, xs, expert_W, expert_b, gs):
    grid_spec = pltpu.PrefetchScalarGridSpec(
        num_scalar_prefetch=1,
        grid=(NBLK,),
        in_specs=[
            pl.BlockSpec((BLK, D), lambda i, be_s: (i, 0)),
            pl.BlockSpec((1, D, D), lambda i, be_s: (be_s[i], 0, 0)),
            pl.BlockSpec((1, 1, D), lambda i, be_s: (be_s[i], 0, 0)),
            pl.BlockSpec((BLK, GW), lambda i, be_s: (i, 0)),
        ],
        out_specs=pl.BlockSpec((BLK, D), lambda i, be_s: (i, 0)),
    )
    return pl.pallas_call(
        _gmm_body,
        grid_spec=grid_spec,
        out_shape=jax.ShapeDtypeStruct((PAD, D), jnp.float32),
    )(be, xs, expert_W, expert_b.reshape(NE, 1, D), gs)


def _combine(ys, pcat):
    mesh = plsc.VectorSubcoreMesh(core_axis_name="c", subcore_axis_name="s")

    @functools.partial(
        pl.kernel, mesh=mesh,
        out_type=jax.ShapeDtypeStruct((NT, D), jnp.float32),
        scratch_types=[
            pltpu.VMEM((2 * CHE, D), jnp.float32),
            pltpu.VMEM((2 * CHE, D), jnp.float32),
            pltpu.VMEM((CHE, D), jnp.float32),
            pltpu.VMEM((CHE, D), jnp.float32),
            pltpu.VMEM((NCHE, 2 * CHE), jnp.int32),
            pltpu.SemaphoreType.DMA,
            pltpu.SemaphoreType.DMA,
            pltpu.SemaphoreType.DMA,
            pltpu.SemaphoreType.DMA,
        ],
    )
    def body(y_hbm, pc_hbm, out_hbm, rb0, rb1, ob0, ob1,
             pcall, gsem0, gsem1, osem0, osem1):
        wid = lax.axis_index("s") * NC + lax.axis_index("c")
        t0 = wid * TPW
        r0 = wid * NCHE
        pltpu.sync_copy(pc_hbm.at[pl.ds(r0, NCHE)], pcall)

        rbufs = (rb0, rb1)
        obufs = (ob0, ob1)
        gsems = (gsem0, gsem1)
        osems = (osem0, osem1)
        gathers = [None, None]
        owrite = [None, None]
        gathers[0] = pltpu.async_copy(y_hbm.at[pcall.at[0]], rb0, gsem0)
        for j in range(NCHE):
            b = j % 2
            gathers[b].wait()
            if j + 1 < NCHE:
                gathers[1 - b] = pltpu.async_copy(
                    y_hbm.at[pcall.at[j + 1]], rbufs[1 - b], gsems[1 - b])
            rb = rbufs[b]
            ob = obufs[b]
            if owrite[b] is not None:
                owrite[b].wait()
            def row(r, carry):
                def col(c, carry2):
                    sl = pl.ds(c * 16, 16)
                    ob[r, sl] = rb[r, sl] + rb[r + CHE, sl]
                    return carry2

                lax.fori_loop(0, D // 16, col, 0)
                return carry

            lax.fori_loop(0, CHE, row, 0)
            owrite[b] = pltpu.async_copy(
                ob, out_hbm.at[pl.ds(t0 + j * CHE, CHE)], osems[b])
        for ow in owrite:
            if ow is not None:
                ow.wait()

    return body(ys, pcat)


def kernel(x, expert_W, expert_b, router_W, router_b):
    i0, i1, g0b, g1b = _router(x, router_W, router_b)
    e0_2d = i0.reshape(NT).reshape(_G // 2, _G)
    e1_2d = i1.reshape(NT).reshape(_G // 2, _G)
    p0_2d, p1_2d, be2d = _bookkeeping(e0_2d, e1_2d)
    p0 = p0_2d.reshape(NT)
    p1 = p1_2d.reshape(NT)
    be = be2d.reshape(1024)[:NBLK]
    p0r = p0.reshape(NT // CH, CH)
    p1r = p1.reshape(NT // CH, CH)
    xs, gs = _dispatch(x, p0r, p1r, g0b, g1b)
    ys = _grouped_matmul(be, xs, expert_W, expert_b, gs)
    pcat = jnp.concatenate(
        [p0.reshape(NT // CHE, CHE), p1.reshape(NT // CHE, CHE)], axis=1)
    return _combine(ys, pcat)
